# Initial kernel scaffold; baseline (speedup 1.0000x reference)
import jax
import jax.numpy as jnp
from jax.experimental import pallas as pl


def kernel(p1, x1, o1, p2, x2, o2, W1, b1, g1, be1, W2, b2, g2, be2, csW1, csb1, csg, csbe, csW2, csb2):
    raise NotImplementedError("write your pallas kernel here")


# speedup vs baseline: 4.4850x; 4.4850x over previous
import functools
import math

import jax
import jax.numpy as jnp
from jax import lax
from jax.experimental import pallas as pl
from jax.experimental.pallas import tpu as pltpu
from jax.experimental.pallas import tpu_sc as plsc

NC = 2
NS = 16
NW = NC * NS
L = 16
_S = 1.0 / math.sqrt(1.0 + 1e-5)


def _rup(x, mult):
    return (x + mult - 1) // mult * mult


def _lin_body(x_ref, w_ref, b_ref, g_ref, be_ref, o_ref):
    acc = jnp.dot(x_ref[...], w_ref[...], preferred_element_type=jnp.float32)
    o_ref[...] = jnp.maximum(
        (acc + b_ref[...]) * (g_ref[...] * _S) + be_ref[...], 0.0)


def _lin1_call(x1, W1, b1, g1, be1):
    n, K = x1.shape
    C = W1.shape[1]
    RB = 2000
    return pl.pallas_call(
        _lin_body,
        grid=(n // RB,),
        in_specs=[
            pl.BlockSpec((RB, K), lambda i: (i, 0)),
            pl.BlockSpec((K, C), lambda i: (0, 0)),
            pl.BlockSpec((1, C), lambda i: (0, 0)),
            pl.BlockSpec((1, C), lambda i: (0, 0)),
            pl.BlockSpec((1, C), lambda i: (0, 0)),
        ],
        out_specs=pl.BlockSpec((RB, C), lambda i: (i, 0)),
        out_shape=jax.ShapeDtypeStruct((n, C), jnp.float32),
    )(x1, W1, b1.reshape(1, C), g1.reshape(1, C), be1.reshape(1, C))


def _dense2_body(x_ref, w2_ref, b2_ref, g2_ref, be2_ref, wc_ref, bc_ref,
                 l2t_ref, a_ref):
    x = x_ref[...]
    acc = jnp.dot(x, w2_ref[...], preferred_element_type=jnp.float32)
    lin2 = jnp.maximum(
        (acc + b2_ref[...]) * (g2_ref[...] * _S) + be2_ref[...], 0.0)
    l2t_ref[...] = lin2.T
    a_ref[...] = jnp.dot(x, wc_ref[...],
                         preferred_element_type=jnp.float32) + bc_ref[...]


def _dense2_call(x2p, W2, b2, g2, be2, Wc, bc):
    Mp, K = x2p.shape
    C = W2.shape[1]
    Ca = Wc.shape[1]
    RB = 1024
    return pl.pallas_call(
        _dense2_body,
        grid=(Mp // RB,),
        in_specs=[
            pl.BlockSpec((RB, K), lambda i: (i, 0)),
            pl.BlockSpec((K, C), lambda i: (0, 0)),
            pl.BlockSpec((1, C), lambda i: (0, 0)),
            pl.BlockSpec((1, C), lambda i: (0, 0)),
            pl.BlockSpec((1, C), lambda i: (0, 0)),
            pl.BlockSpec((K, Ca), lambda i: (0, 0)),
            pl.BlockSpec((1, Ca), lambda i: (0, 0)),
        ],
        out_specs=[
            pl.BlockSpec((C, RB), lambda i: (0, i)),
            pl.BlockSpec((RB, Ca), lambda i: (i, 0)),
        ],
        out_shape=[
            jax.ShapeDtypeStruct((C, Mp), jnp.float32),
            jax.ShapeDtypeStruct((Mp, Ca), jnp.float32),
        ],
    )(x2p, W2, b2.reshape(1, C), g2.reshape(1, C), be2.reshape(1, C),
      Wc, bc.reshape(1, Ca))


def _shrink_body(a_ref, px_ref, py_ref, pz_ref, v_ref, g_ref, be_ref, w2_ref,
                 b2_ref, o_ref):
    sg = g_ref[...] * _S
    a2 = a_ref[...] * sg + be_ref[...]
    v0 = v_ref[0:1, :] * sg
    v1 = v_ref[1:2, :] * sg
    v2 = v_ref[2:3, :] * sg
    w2 = w2_ref[...]
    cols = []
    for kk in range(L):
        b = (px_ref[:, kk:kk + 1] * v0 + py_ref[:, kk:kk + 1] * v1
             + pz_ref[:, kk:kk + 1] * v2)
        h = jnp.maximum(a2 + b, 0.0)
        cols.append(jnp.sum(h * w2, axis=1, keepdims=True))
    o_ref[...] = jnp.concatenate(cols, axis=1) + b2_ref[0, 0]


def _shrink_call(A, prx, pry, prz, V3p, csg, csbe, w2row, csb2):
    Mp, C = A.shape
    RB = 1024
    return pl.pallas_call(
        _shrink_body,
        grid=(Mp // RB,),
        in_specs=[
            pl.BlockSpec((RB, C), lambda i: (i, 0)),
            pl.BlockSpec((RB, L), lambda i: (i, 0)),
            pl.BlockSpec((RB, L), lambda i: (i, 0)),
            pl.BlockSpec((RB, L), lambda i: (i, 0)),
            pl.BlockSpec((8, C), lambda i: (0, 0)),
            pl.BlockSpec((1, C), lambda i: (0, 0)),
            pl.BlockSpec((1, C), lambda i: (0, 0)),
            pl.BlockSpec((1, C), lambda i: (0, 0)),
            pl.BlockSpec((1, 1), lambda i: (0, 0)),
        ],
        out_specs=pl.BlockSpec((RB, L), lambda i: (i, 0)),
        out_shape=jax.ShapeDtypeStruct((Mp, L), jnp.float32),
    )(A, prx, pry, prz, V3p, csg.reshape(1, C), csbe.reshape(1, C),
      w2row, csb2.reshape(1, 1))


def _final_body(l1_ref, upt_ref, o_ref):
    o_ref[...] = l1_ref[...] + upt_ref[...].T


def _final_call(lin1, upT):
    n, C = lin1.shape
    RB = 2048
    return pl.pallas_call(
        _final_body,
        grid=(n // RB,),
        in_specs=[
            pl.BlockSpec((RB, C), lambda i: (i, 0)),
            pl.BlockSpec((C, RB), lambda i: (0, i)),
        ],
        out_specs=pl.BlockSpec((RB, C), lambda i: (i, 0)),
        out_shape=jax.ShapeDtypeStruct((n, C), jnp.float32),
    )(lin1, upT)


def _bf16r(x):
    u = plsc.bitcast(x, jnp.int32)
    b = 32767 + ((u >> 16) & 1)
    return plsc.bitcast((u + b) & (-65536), jnp.float32)


def _make_knn(n1, PER, Mp):
    mesh = plsc.VectorSubcoreMesh(core_axis_name="c", subcore_axis_name="s")
    NSTEP = n1 // L
    R = 5
    LOWM = 0x7FF

    @functools.partial(
        pl.kernel, mesh=mesh,
        compiler_params=pltpu.CompilerParams(needs_layout_passes=False),
        out_type=[
            jax.ShapeDtypeStruct((Mp * L,), jnp.int32),
            jax.ShapeDtypeStruct((Mp * L,), jnp.float32),
            jax.ShapeDtypeStruct((Mp * L,), jnp.float32),
            jax.ShapeDtypeStruct((Mp * L,), jnp.float32),
        ],
        scratch_types=[
            pltpu.VMEM((n1,), jnp.float32),
            pltpu.VMEM((n1,), jnp.float32),
            pltpu.VMEM((n1,), jnp.float32),
            pltpu.VMEM((n1,), jnp.float32),
            pltpu.VMEM((3 * PER,), jnp.float32),
            pltpu.VMEM((PER * L,), jnp.int32),
            pltpu.VMEM((PER * L,), jnp.float32),
            pltpu.VMEM((PER * L,), jnp.float32),
            pltpu.VMEM((PER * L,), jnp.float32),
        ],
    )
    def knn(p1x_hbm, p1y_hbm, p1z_hbm, p2b_hbm, oidx, oprx, opry, oprz,
            p1a, p1b, p1c, p1q, qv, bidx, bx, by, bz):
        wid = lax.axis_index("s") * NC + lax.axis_index("c")
        pltpu.sync_copy(p1x_hbm, p1a)
        pltpu.sync_copy(p1y_hbm, p1b)
        pltpu.sync_copy(p1z_hbm, p1c)
        pltpu.sync_copy(p2b_hbm.at[pl.ds(wid * 3 * PER, 3 * PER)], qv)

        def sq_step(j, _):
            o = j * L
            px = p1a[pl.ds(o, L)]
            py = p1b[pl.ds(o, L)]
            pz = p1c[pl.ds(o, L)]
            p1q[pl.ds(o, L)] = px * px + py * py + pz * pz
            p1a[pl.ds(o, L)] = -2.0 * _bf16r(px)
            p1b[pl.ds(o, L)] = -2.0 * _bf16r(py)
            p1c[pl.ds(o, L)] = -2.0 * _bf16r(pz)
            return 0

        lax.fori_loop(0, NSTEP, sq_step, 0)

        iota = lax.iota(jnp.int32, L)
        inf16 = jnp.full((L,), jnp.inf, jnp.float32)

        def per_query(i, _):
            iv = jnp.full((L,), i, jnp.int32)
            qx = plsc.load_gather(qv, [iv])
            qy = plsc.load_gather(qv, [iv + PER])
            qz = plsc.load_gather(qv, [iv + 2 * PER])
            qsq = qx * qx + qy * qy + qz * qz
            qxb = _bf16r(qx)
            qyb = _bf16r(qy)
            qzb = _bf16r(qz)

            def dvec(s):
                o = s * L
                return ((p1q[pl.ds(o, L)] + qsq)
                        + (qxb * p1a[pl.ds(o, L)] + qyb * p1b[pl.ds(o, L)]
                           + qzb * p1c[pl.ds(o, L)]))

            def step_fn(s, A):
                u = plsc.bitcast(dvec(s), jnp.int32)
                t = plsc.bitcast((u & (~LOWM)) | s, jnp.float32)
                out = []
                for r in range(R):
                    out.append(jnp.minimum(A[r], t))
                    t = jnp.maximum(A[r], t)
                return tuple(out)

            A = lax.fori_loop(0, NSTEP, step_fn, (inf16,) * R)

            M_d = None
            M_i = None
            for r in range(R):
                u = plsc.bitcast(A[r], jnp.int32)
                ci = (u & LOWM) * L + iota
                de = ((plsc.load_gather(p1q, [ci]) + qsq)
                      + (qxb * plsc.load_gather(p1a, [ci])
                         + qyb * plsc.load_gather(p1b, [ci])
                         + qzb * plsc.load_gather(p1c, [ci])))
                Cs, Ci = plsc.sort_key_val(de, ci)
                if r == 0:
                    M_d, M_i = Cs, Ci
                else:
                    rC = lax.rev(Cs, (0,))
                    rCi = lax.rev(Ci, (0,))
                    mm = M_d <= rC
                    nd = jnp.where(mm, M_d, rC)
                    ni = jnp.where(mm, M_i, rCi)
                    M_d, M_i = plsc.sort_key_val(nd, ni)

            tau = jnp.broadcast_to(jnp.max(M_d), (L,))
            cl = plsc.bitcast(plsc.bitcast(A[R - 1], jnp.int32) & (~LOWM),
                              jnp.float32)
            bad = jnp.any(cl - 0.002 * jnp.abs(cl) < tau)

            def exact(_):
                d0 = dvec(0)
                Ad, Ai = plsc.sort_key_val(d0, iota)
                t0 = jnp.broadcast_to(jnp.max(Ad), (L,))

                def estep(s, carry):
                    Ad, Ai, t = carry
                    d = dvec(s)
                    hit = jnp.any(d < t)

                    def merge(c):
                        Ad, Ai, _ = c
                        Cs, Ci = plsc.sort_key_val(d, iota + s * L)
                        rC = lax.rev(Cs, (0,))
                        rCi = lax.rev(Ci, (0,))
                        mm = Ad <= rC
                        nd = jnp.where(mm, Ad, rC)
                        ni = jnp.where(mm, Ai, rCi)
                        Ad2, Ai2 = plsc.sort_key_val(nd, ni)
                        return (Ad2, Ai2,
                                jnp.broadcast_to(jnp.max(Ad2), (L,)))

                    return lax.cond(hit, merge, lambda c: c, (Ad, Ai, t))

                Ad, Ai, _ = lax.fori_loop(1, NSTEP, estep, (Ad, Ai, t0))
                return Ad, Ai

            M_d, M_i = lax.cond(bad, exact, lambda _: (M_d, M_i), 0)
            bidx[pl.ds(i * L, L)] = M_i
            return 0

        lax.fori_loop(0, PER, per_query, 0)

        pltpu.sync_copy(p1x_hbm, p1a)
        pltpu.sync_copy(p1y_hbm, p1b)
        pltpu.sync_copy(p1z_hbm, p1c)

        def pr_query(i, _):
            iv = jnp.full((L,), i, jnp.int32)
            qx = plsc.load_gather(qv, [iv])
            qy = plsc.load_gather(qv, [iv + PER])
            qz = plsc.load_gather(qv, [iv + 2 * PER])
            o = i * L
            idx = bidx[pl.ds(o, L)]
            bx[pl.ds(o, L)] = plsc.load_gather(p1a, [idx]) - qx
            by[pl.ds(o, L)] = plsc.load_gather(p1b, [idx]) - qy
            bz[pl.ds(o, L)] = plsc.load_gather(p1c, [idx]) - qz
            return 0

        lax.fori_loop(0, PER, pr_query, 0)

        base = wid * (PER * L)
        pltpu.sync_copy(bidx, oidx.at[pl.ds(base, PER * L)])
        pltpu.sync_copy(bx, oprx.at[pl.ds(base, PER * L)])
        pltpu.sync_copy(by, opry.at[pl.ds(base, PER * L)])
        pltpu.sync_copy(bz, oprz.at[pl.ds(base, PER * L)])

    return knn


def _make_exden(n1, m, Mp, N1p):
    mesh = plsc.VectorSubcoreMesh(core_axis_name="c", subcore_axis_name="s")
    PERB = Mp // NS

    @functools.partial(
        pl.kernel, mesh=mesh,
        compiler_params=pltpu.CompilerParams(needs_layout_passes=False),
        out_type=[
            jax.ShapeDtypeStruct((NC * NS * N1p,), jnp.float32),
            jax.ShapeDtypeStruct((NC * Mp * L,), jnp.float32),
        ],
        scratch_types=[
            pltpu.VMEM((PERB * L,), jnp.float32),
            pltpu.VMEM((PERB * L,), jnp.int32),
            pltpu.VMEM((N1p,), jnp.float32),
        ],
    )
    def exden(shrink_hbm, kif_hbm, part_hbm, ex_hbm, sh, kf, buf):
        cid = lax.axis_index("c")
        sid = lax.axis_index("s")

        def mx_chunk(c, mv):
            pltpu.sync_copy(shrink_hbm.at[pl.ds(c * (PERB * L), PERB * L)], sh)

            def mx_step(i, mv):
                return jnp.maximum(mv, sh[pl.ds(i * L, L)])

            return lax.fori_loop(0, PERB, mx_step, mv)

        mv = lax.fori_loop(0, NS, mx_chunk,
                           jnp.full((L,), -jnp.inf, jnp.float32))
        gmax = jnp.broadcast_to(jnp.max(mv), (L,))

        base = sid * (PERB * L)
        pltpu.sync_copy(shrink_hbm.at[pl.ds(base, PERB * L)], sh)
        pltpu.sync_copy(kif_hbm.at[pl.ds(base, PERB * L)], kf)

        def z_step(j, _):
            buf[pl.ds(j * L, L)] = jnp.zeros((L,), jnp.float32)
            return 0

        lax.fori_loop(0, N1p // L, z_step, 0)
        start_q = sid * PERB

        def ex_step(i, _):
            o = i * L
            e = jnp.exp(sh[pl.ds(o, L)] - gmax)
            e = e * ((start_q + i) < m).astype(jnp.float32)
            sh[pl.ds(o, L)] = e
            plsc.addupdate_scatter(buf, [kf[pl.ds(o, L)]], e)
            return 0

        lax.fori_loop(0, PERB, ex_step, 0)
        pltpu.sync_copy(buf, part_hbm.at[pl.ds((cid * NS + sid) * N1p, N1p)])
        pltpu.sync_copy(sh, ex_hbm.at[pl.ds(cid * (Mp * L) + base, PERB * L)])

    return exden


def _make_probs(m, Mp, N1p):
    mesh = plsc.VectorSubcoreMesh(core_axis_name="c", subcore_axis_name="s")
    PERB = Mp // NS

    @functools.partial(
        pl.kernel, mesh=mesh,
        compiler_params=pltpu.CompilerParams(needs_layout_passes=False),
        out_type=[
            jax.ShapeDtypeStruct((NC * Mp * L,), jnp.float32),
        ],
        scratch_types=[
            pltpu.VMEM((N1p,), jnp.float32),
            pltpu.VMEM((N1p,), jnp.float32),
            pltpu.VMEM((PERB * L,), jnp.float32),
            pltpu.VMEM((PERB * L,), jnp.int32),
        ],
    )
    def probs(part_hbm, ex_hbm, kif_hbm, prob_hbm, acc, tmp, exv, kf):
        cid = lax.axis_index("c")
        sid = lax.axis_index("s")
        pbase = cid * NS * N1p
        pltpu.sync_copy(part_hbm.at[pl.ds(pbase, N1p)], acc)

        def red_step(c, _):
            pltpu.sync_copy(part_hbm.at[pl.ds(pbase + c * N1p, N1p)], tmp)

            def add_step(j, _):
                o = j * L
                acc[pl.ds(o, L)] = acc[pl.ds(o, L)] + tmp[pl.ds(o, L)]
                return 0

            lax.fori_loop(0, N1p // L, add_step, 0)
            return 0

        lax.fori_loop(1, NS, red_step, 0)

        base = sid * (PERB * L)
        pltpu.sync_copy(ex_hbm.at[pl.ds(cid * (Mp * L) + base, PERB * L)], exv)
        pltpu.sync_copy(kif_hbm.at[pl.ds(base, PERB * L)], kf)

        def pr_step(i, _):
            o = i * L
            den = plsc.load_gather(acc, [kf[pl.ds(o, L)]]) + 1e-16
            exv[pl.ds(o, L)] = exv[pl.ds(o, L)] / den
            return 0

        lax.fori_loop(0, PERB, pr_step, 0)
        pltpu.sync_copy(exv, prob_hbm.at[pl.ds(cid * (Mp * L) + base,
                                               PERB * L)])

    return probs


def _make_upscatter(Mp, N1p, C):
    mesh = plsc.VectorSubcoreMesh(core_axis_name="c", subcore_axis_name="s")
    CPW = C // NW
    CPP = CPW // 2
    QCH = 256
    NCH = Mp // QCH

    @functools.partial(
        pl.kernel, mesh=mesh,
        compiler_params=pltpu.CompilerParams(needs_layout_passes=False),
        out_type=[
            jax.ShapeDtypeStruct((C * N1p,), jnp.float32),
        ],
        scratch_types=[
            pltpu.VMEM((CPP * N1p,), jnp.float32),
            pltpu.VMEM((QCH * L,), jnp.int32),
            pltpu.VMEM((QCH * L,), jnp.float32),
            pltpu.VMEM((CPP * Mp,), jnp.float32),
        ],
    )
    def upscatter(prob_hbm, kif_hbm, l2t_hbm, upt_hbm, buf, ck, cp, l2):
        cid = lax.axis_index("c")
        sid = lax.axis_index("s")
        wid = sid * NC + cid
        for p in range(2):
            gch = wid * CPW + p * CPP

            def ze_step(j, _):
                buf[pl.ds(j * L, L)] = jnp.zeros((L,), jnp.float32)
                return 0

            lax.fori_loop(0, (CPP * N1p) // L, ze_step, 0)
            pltpu.sync_copy(l2t_hbm.at[pl.ds(gch * Mp, CPP * Mp)], l2)

            def ch_step(c, _):
                cb = c * (QCH * L)
                pltpu.sync_copy(kif_hbm.at[pl.ds(cb, QCH * L)], ck)
                pltpu.sync_copy(
                    prob_hbm.at[pl.ds(cid * (Mp * L) + cb, QCH * L)], cp)

                def q_step(i, _):
                    o = i * L
                    kr = ck[pl.ds(o, L)]
                    pr = cp[pl.ds(o, L)]
                    qcol = jnp.full((L,), c * QCH + i, jnp.int32)
                    for ch in range(CPP):
                        val = plsc.load_gather(l2, [qcol + ch * Mp])
                        plsc.addupdate_scatter(
                            buf, [kr + ch * N1p], pr * val)
                    return 0

                lax.fori_loop(0, QCH, q_step, 0)
                return 0

            lax.fori_loop(0, NCH, ch_step, 0)
            for ch in range(CPP):
                pltpu.sync_copy(buf.at[pl.ds(ch * N1p, N1p)],
                                upt_hbm.at[pl.ds((gch + ch) * N1p, N1p)])

    return upscatter


def kernel(p1, x1, o1, p2, x2, o2, W1, b1, g1, be1, W2, b2, g2, be2,
           csW1, csb1, csg, csbe, csW2, csb2):
    n1 = p1.shape[0]
    m = p2.shape[0]
    C = x1.shape[1]
    Ci = x2.shape[1]

    PER = _rup(-(-m // NW), 8)
    Mp = PER * NW
    DSTP = _rup(_rup(n1, 2048) // NW, 8)
    N1p = DSTP * NW

    p1x = jnp.asarray(p1[:, 0], jnp.float32)
    p1y = jnp.asarray(p1[:, 1], jnp.float32)
    p1z = jnp.asarray(p1[:, 2], jnp.float32)
    p2Tp = jnp.pad(p2.T, ((0, 0), (0, Mp - m)))
    p2b = p2Tp.reshape(3, NW, PER).transpose(1, 0, 2).reshape(NW * 3 * PER)
    x2p = jnp.pad(x2, ((0, Mp - m), (0, 0)))
    V3p = jnp.pad(csW1[:3], ((0, 5), (0, 0)))
    w2row = csW2.reshape(1, Ci)

    lin1 = _lin1_call(x1, W1, b1, g1, be1)
    lin2T, A = _dense2_call(x2p, W2, b2, g2, be2, csW1[3:], csb1)

    knn = _make_knn(n1, PER, Mp)
    kif1, prx1, pry1, prz1 = knn(p1x, p1y, p1z, p2b)
    prx = prx1.reshape(Mp, L)
    pry = pry1.reshape(Mp, L)
    prz = prz1.reshape(Mp, L)

    shrink = _shrink_call(A, prx, pry, prz, V3p, csg, csbe, w2row, csb2)

    shrink1 = shrink.reshape(Mp * L)
    partials, exs = _make_exden(n1, m, Mp, N1p)(shrink1, kif1)
    probs, = _make_probs(m, Mp, N1p)(partials, exs, kif1)
    upTf, = _make_upscatter(Mp, N1p, C)(probs, kif1, lin2T.reshape(C * Mp))
    upT = upTf.reshape(C, N1p)

    lin1p = jnp.pad(lin1, ((0, N1p - n1), (0, 0)))
    return _final_call(lin1p, upT)[:n1]


# speedup vs baseline: 5.7319x; 1.2780x over previous
import functools
import math

import jax
import jax.numpy as jnp
from jax import lax
from jax.experimental import pallas as pl
from jax.experimental.pallas import tpu as pltpu
from jax.experimental.pallas import tpu_sc as plsc

NC = 2
NS = 16
NW = NC * NS
L = 16
_S = 1.0 / math.sqrt(1.0 + 1e-5)


def _rup(x, mult):
    return (x + mult - 1) // mult * mult


def _lin_body(x_ref, w_ref, b_ref, g_ref, be_ref, o_ref):
    acc = jnp.dot(x_ref[...], w_ref[...], preferred_element_type=jnp.float32)
    o_ref[...] = jnp.maximum(
        (acc + b_ref[...]) * (g_ref[...] * _S) + be_ref[...], 0.0)


def _lin1_call(x1, W1, b1, g1, be1):
    n, K = x1.shape
    C = W1.shape[1]
    RB = 2000
    return pl.pallas_call(
        _lin_body,
        grid=(n // RB,),
        in_specs=[
            pl.BlockSpec((RB, K), lambda i: (i, 0)),
            pl.BlockSpec((K, C), lambda i: (0, 0)),
            pl.BlockSpec((1, C), lambda i: (0, 0)),
            pl.BlockSpec((1, C), lambda i: (0, 0)),
            pl.BlockSpec((1, C), lambda i: (0, 0)),
        ],
        out_specs=pl.BlockSpec((RB, C), lambda i: (i, 0)),
        out_shape=jax.ShapeDtypeStruct((n, C), jnp.float32),
    )(x1, W1, b1.reshape(1, C), g1.reshape(1, C), be1.reshape(1, C))


def _dense2_body(x_ref, w2_ref, b2_ref, g2_ref, be2_ref, wc_ref, bc_ref,
                 l2t_ref, a_ref):
    x = x_ref[...]
    acc = jnp.dot(x, w2_ref[...], preferred_element_type=jnp.float32)
    lin2 = jnp.maximum(
        (acc + b2_ref[...]) * (g2_ref[...] * _S) + be2_ref[...], 0.0)
    l2t_ref[...] = lin2.T
    a_ref[...] = jnp.dot(x, wc_ref[...],
                         preferred_element_type=jnp.float32) + bc_ref[...]


def _dense2_call(x2p, W2, b2, g2, be2, Wc, bc):
    Mp, K = x2p.shape
    C = W2.shape[1]
    Ca = Wc.shape[1]
    RB = 1024
    return pl.pallas_call(
        _dense2_body,
        grid=(Mp // RB,),
        in_specs=[
            pl.BlockSpec((RB, K), lambda i: (i, 0)),
            pl.BlockSpec((K, C), lambda i: (0, 0)),
            pl.BlockSpec((1, C), lambda i: (0, 0)),
            pl.BlockSpec((1, C), lambda i: (0, 0)),
            pl.BlockSpec((1, C), lambda i: (0, 0)),
            pl.BlockSpec((K, Ca), lambda i: (0, 0)),
            pl.BlockSpec((1, Ca), lambda i: (0, 0)),
        ],
        out_specs=[
            pl.BlockSpec((C, RB), lambda i: (0, i)),
            pl.BlockSpec((RB, Ca), lambda i: (i, 0)),
        ],
        out_shape=[
            jax.ShapeDtypeStruct((C, Mp), jnp.float32),
            jax.ShapeDtypeStruct((Mp, Ca), jnp.float32),
        ],
    )(x2p, W2, b2.reshape(1, C), g2.reshape(1, C), be2.reshape(1, C),
      Wc, bc.reshape(1, Ca))


def _shrink_body(a_ref, px_ref, py_ref, pz_ref, v_ref, g_ref, be_ref, w2_ref,
                 b2_ref, o_ref):
    sg = g_ref[...] * _S
    a2 = a_ref[...] * sg + be_ref[...]
    v0 = v_ref[0:1, :] * sg
    v1 = v_ref[1:2, :] * sg
    v2 = v_ref[2:3, :] * sg
    w2 = w2_ref[...]
    cols = []
    for kk in range(L):
        b = (px_ref[:, kk:kk + 1] * v0 + py_ref[:, kk:kk + 1] * v1
             + pz_ref[:, kk:kk + 1] * v2)
        h = jnp.maximum(a2 + b, 0.0)
        cols.append(jnp.sum(h * w2, axis=1, keepdims=True))
    o_ref[...] = jnp.concatenate(cols, axis=1) + b2_ref[0, 0]


def _shrink_call(A, prx, pry, prz, V3p, csg, csbe, w2row, csb2):
    Mp, C = A.shape
    RB = 1024
    return pl.pallas_call(
        _shrink_body,
        grid=(Mp // RB,),
        in_specs=[
            pl.BlockSpec((RB, C), lambda i: (i, 0)),
            pl.BlockSpec((RB, L), lambda i: (i, 0)),
            pl.BlockSpec((RB, L), lambda i: (i, 0)),
            pl.BlockSpec((RB, L), lambda i: (i, 0)),
            pl.BlockSpec((8, C), lambda i: (0, 0)),
            pl.BlockSpec((1, C), lambda i: (0, 0)),
            pl.BlockSpec((1, C), lambda i: (0, 0)),
            pl.BlockSpec((1, C), lambda i: (0, 0)),
            pl.BlockSpec((1, 1), lambda i: (0, 0)),
        ],
        out_specs=pl.BlockSpec((RB, L), lambda i: (i, 0)),
        out_shape=jax.ShapeDtypeStruct((Mp, L), jnp.float32),
    )(A, prx, pry, prz, V3p, csg.reshape(1, C), csbe.reshape(1, C),
      w2row, csb2.reshape(1, 1))


def _final_body(l1_ref, upt_ref, o_ref):
    o_ref[...] = l1_ref[...] + upt_ref[...].T


def _final_call(lin1, upT):
    n, C = lin1.shape
    RB = 2048
    return pl.pallas_call(
        _final_body,
        grid=(n // RB,),
        in_specs=[
            pl.BlockSpec((RB, C), lambda i: (i, 0)),
            pl.BlockSpec((C, RB), lambda i: (0, i)),
        ],
        out_specs=pl.BlockSpec((RB, C), lambda i: (i, 0)),
        out_shape=jax.ShapeDtypeStruct((n, C), jnp.float32),
    )(lin1, upT)


def _bf16r(x):
    u = plsc.bitcast(x, jnp.int32)
    b = 32767 + ((u >> 16) & 1)
    return plsc.bitcast((u + b) & (-65536), jnp.float32)


def _make_knn(n1, PER, Mp):
    mesh = plsc.VectorSubcoreMesh(core_axis_name="c", subcore_axis_name="s")
    NSTEP = n1 // L
    R = 6
    RFB = 16
    LOWM = 0x7FF

    @functools.partial(
        pl.kernel, mesh=mesh,
        compiler_params=pltpu.CompilerParams(needs_layout_passes=False),
        out_type=[
            jax.ShapeDtypeStruct((Mp * L,), jnp.int32),
            jax.ShapeDtypeStruct((Mp * L,), jnp.float32),
            jax.ShapeDtypeStruct((Mp * L,), jnp.float32),
            jax.ShapeDtypeStruct((Mp * L,), jnp.float32),
        ],
        scratch_types=[
            pltpu.VMEM((n1,), jnp.float32),
            pltpu.VMEM((n1,), jnp.float32),
            pltpu.VMEM((n1,), jnp.float32),
            pltpu.VMEM((n1,), jnp.float32),
            pltpu.VMEM((3 * PER,), jnp.float32),
            pltpu.VMEM((PER * L,), jnp.int32),
            pltpu.VMEM((PER * L,), jnp.float32),
            pltpu.VMEM((PER * L,), jnp.float32),
            pltpu.VMEM((PER * L,), jnp.float32),
        ],
    )
    def knn(p1x_hbm, p1y_hbm, p1z_hbm, p2b_hbm, oidx, oprx, opry, oprz,
            p1a, p1b, p1c, p1q, qv, bidx, bx, by, bz):
        wid = lax.axis_index("s") * NC + lax.axis_index("c")
        pltpu.sync_copy(p1x_hbm, p1a)
        pltpu.sync_copy(p1y_hbm, p1b)
        pltpu.sync_copy(p1z_hbm, p1c)
        pltpu.sync_copy(p2b_hbm.at[pl.ds(wid * 3 * PER, 3 * PER)], qv)

        def sq_step(j, _):
            o = j * L
            px = p1a[pl.ds(o, L)]
            py = p1b[pl.ds(o, L)]
            pz = p1c[pl.ds(o, L)]
            p1q[pl.ds(o, L)] = px * px + py * py + pz * pz
            p1a[pl.ds(o, L)] = -2.0 * _bf16r(px)
            p1b[pl.ds(o, L)] = -2.0 * _bf16r(py)
            p1c[pl.ds(o, L)] = -2.0 * _bf16r(pz)
            return 0

        lax.fori_loop(0, NSTEP, sq_step, 0)

        iota = lax.iota(jnp.int32, L)
        inf16 = jnp.full((L,), jnp.inf, jnp.float32)

        def per_query(i, _):
            iv = jnp.full((L,), i, jnp.int32)
            qx = plsc.load_gather(qv, [iv])
            qy = plsc.load_gather(qv, [iv + PER])
            qz = plsc.load_gather(qv, [iv + 2 * PER])
            qsq = qx * qx + qy * qy + qz * qz
            qxb = _bf16r(qx)
            qyb = _bf16r(qy)
            qzb = _bf16r(qz)

            def dvec(s):
                o = s * L
                return ((p1q[pl.ds(o, L)] + qsq)
                        + (qxb * p1a[pl.ds(o, L)] + qyb * p1b[pl.ds(o, L)]
                           + qzb * p1c[pl.ds(o, L)]))

            def step_fn(s, A):
                u = plsc.bitcast(dvec(s), jnp.int32)
                t = plsc.bitcast((u & (~LOWM)) | s, jnp.float32)
                out = []
                for r in range(R):
                    out.append(jnp.minimum(A[r], t))
                    t = jnp.maximum(A[r], t)
                return tuple(out)

            A = lax.fori_loop(0, NSTEP, step_fn, (inf16,) * R)

            M_d = None
            M_i = None
            for r in range(R):
                u = plsc.bitcast(A[r], jnp.int32)
                ci = (u & LOWM) * L + iota
                de = ((plsc.load_gather(p1q, [ci]) + qsq)
                      + (qxb * plsc.load_gather(p1a, [ci])
                         + qyb * plsc.load_gather(p1b, [ci])
                         + qzb * plsc.load_gather(p1c, [ci])))
                Cs, Ci = plsc.sort_key_val(de, ci)
                if r == 0:
                    M_d, M_i = Cs, Ci
                else:
                    rC = lax.rev(Cs, (0,))
                    rCi = lax.rev(Ci, (0,))
                    mm = M_d <= rC
                    nd = jnp.where(mm, M_d, rC)
                    ni = jnp.where(mm, M_i, rCi)
                    M_d, M_i = plsc.sort_key_val(nd, ni)

            tau = jnp.broadcast_to(jnp.max(M_d), (L,))
            cl = plsc.bitcast(plsc.bitcast(A[R - 1], jnp.int32) & (~LOWM),
                              jnp.float32)
            bad = jnp.any(cl - 0.002 * jnp.abs(cl) < tau)

            def exact(_):
                def step2(s, A2):
                    u = plsc.bitcast(dvec(s), jnp.int32)
                    t = plsc.bitcast((u & (~LOWM)) | s, jnp.float32)
                    out = []
                    for r in range(RFB):
                        out.append(jnp.minimum(A2[r], t))
                        t = jnp.maximum(A2[r], t)
                    return tuple(out)

                A2 = lax.fori_loop(0, NSTEP, step2, (inf16,) * RFB)
                Fd = None
                Fi = None
                for r in range(RFB):
                    u = plsc.bitcast(A2[r], jnp.int32)
                    ci = (u & LOWM) * L + iota
                    de = ((plsc.load_gather(p1q, [ci]) + qsq)
                          + (qxb * plsc.load_gather(p1a, [ci])
                             + qyb * plsc.load_gather(p1b, [ci])
                             + qzb * plsc.load_gather(p1c, [ci])))
                    Cs, Ci = plsc.sort_key_val(de, ci)
                    if r == 0:
                        Fd, Fi = Cs, Ci
                    else:
                        rC = lax.rev(Cs, (0,))
                        rCi = lax.rev(Ci, (0,))
                        mm = Fd <= rC
                        nd = jnp.where(mm, Fd, rC)
                        ni = jnp.where(mm, Fi, rCi)
                        Fd, Fi = plsc.sort_key_val(nd, ni)
                return Fd, Fi

            M_d, M_i = lax.cond(bad, exact, lambda _: (M_d, M_i), 0)
            bidx[pl.ds(i * L, L)] = M_i
            return 0

        lax.fori_loop(0, PER, per_query, 0)

        pltpu.sync_copy(p1x_hbm, p1a)
        pltpu.sync_copy(p1y_hbm, p1b)
        pltpu.sync_copy(p1z_hbm, p1c)

        def pr_query(i, _):
            iv = jnp.full((L,), i, jnp.int32)
            qx = plsc.load_gather(qv, [iv])
            qy = plsc.load_gather(qv, [iv + PER])
            qz = plsc.load_gather(qv, [iv + 2 * PER])
            o = i * L
            idx = bidx[pl.ds(o, L)]
            bx[pl.ds(o, L)] = plsc.load_gather(p1a, [idx]) - qx
            by[pl.ds(o, L)] = plsc.load_gather(p1b, [idx]) - qy
            bz[pl.ds(o, L)] = plsc.load_gather(p1c, [idx]) - qz
            return 0

        lax.fori_loop(0, PER, pr_query, 0)

        base = wid * (PER * L)
        pltpu.sync_copy(bidx, oidx.at[pl.ds(base, PER * L)])
        pltpu.sync_copy(bx, oprx.at[pl.ds(base, PER * L)])
        pltpu.sync_copy(by, opry.at[pl.ds(base, PER * L)])
        pltpu.sync_copy(bz, oprz.at[pl.ds(base, PER * L)])

    return knn


def _make_exden(n1, m, Mp, N1p):
    mesh = plsc.VectorSubcoreMesh(core_axis_name="c", subcore_axis_name="s")
    PERB = Mp // NS

    @functools.partial(
        pl.kernel, mesh=mesh,
        compiler_params=pltpu.CompilerParams(needs_layout_passes=False),
        out_type=[
            jax.ShapeDtypeStruct((NC * NS * N1p,), jnp.float32),
            jax.ShapeDtypeStruct((NC * Mp * L,), jnp.float32),
        ],
        scratch_types=[
            pltpu.VMEM((PERB * L,), jnp.float32),
            pltpu.VMEM((PERB * L,), jnp.int32),
            pltpu.VMEM((N1p,), jnp.float32),
        ],
    )
    def exden(shrink_hbm, kif_hbm, part_hbm, ex_hbm, sh, kf, buf):
        cid = lax.axis_index("c")
        sid = lax.axis_index("s")

        def mx_chunk(c, mv):
            pltpu.sync_copy(shrink_hbm.at[pl.ds(c * (PERB * L), PERB * L)], sh)

            def mx_step(i, mv):
                return jnp.maximum(mv, sh[pl.ds(i * L, L)])

            return lax.fori_loop(0, PERB, mx_step, mv)

        mv = lax.fori_loop(0, NS, mx_chunk,
                           jnp.full((L,), -jnp.inf, jnp.float32))
        gmax = jnp.broadcast_to(jnp.max(mv), (L,))

        base = sid * (PERB * L)
        pltpu.sync_copy(shrink_hbm.at[pl.ds(base, PERB * L)], sh)
        pltpu.sync_copy(kif_hbm.at[pl.ds(base, PERB * L)], kf)

        def z_step(j, _):
            buf[pl.ds(j * L, L)] = jnp.zeros((L,), jnp.float32)
            return 0

        lax.fori_loop(0, N1p // L, z_step, 0)
        start_q = sid * PERB

        def ex_step(i, _):
            o = i * L
            e = jnp.exp(sh[pl.ds(o, L)] - gmax)
            e = e * ((start_q + i) < m).astype(jnp.float32)
            sh[pl.ds(o, L)] = e
            plsc.addupdate_scatter(buf, [kf[pl.ds(o, L)]], e)
            return 0

        lax.fori_loop(0, PERB, ex_step, 0)
        pltpu.sync_copy(buf, part_hbm.at[pl.ds((cid * NS + sid) * N1p, N1p)])
        pltpu.sync_copy(sh, ex_hbm.at[pl.ds(cid * (Mp * L) + base, PERB * L)])

    return exden


def _make_probs(m, Mp, N1p):
    mesh = plsc.VectorSubcoreMesh(core_axis_name="c", subcore_axis_name="s")
    PERB = Mp // NS

    @functools.partial(
        pl.kernel, mesh=mesh,
        compiler_params=pltpu.CompilerParams(needs_layout_passes=False),
        out_type=[
            jax.ShapeDtypeStruct((NC * Mp * L,), jnp.float32),
        ],
        scratch_types=[
            pltpu.VMEM((N1p,), jnp.float32),
            pltpu.VMEM((N1p,), jnp.float32),
            pltpu.VMEM((PERB * L,), jnp.float32),
            pltpu.VMEM((PERB * L,), jnp.int32),
        ],
    )
    def probs(part_hbm, ex_hbm, kif_hbm, prob_hbm, acc, tmp, exv, kf):
        cid = lax.axis_index("c")
        sid = lax.axis_index("s")
        pbase = cid * NS * N1p
        pltpu.sync_copy(part_hbm.at[pl.ds(pbase, N1p)], acc)

        def red_step(c, _):
            pltpu.sync_copy(part_hbm.at[pl.ds(pbase + c * N1p, N1p)], tmp)

            def add_step(j, _):
                o = j * L
                acc[pl.ds(o, L)] = acc[pl.ds(o, L)] + tmp[pl.ds(o, L)]
                return 0

            lax.fori_loop(0, N1p // L, add_step, 0)
            return 0

        lax.fori_loop(1, NS, red_step, 0)

        base = sid * (PERB * L)
        pltpu.sync_copy(ex_hbm.at[pl.ds(cid * (Mp * L) + base, PERB * L)], exv)
        pltpu.sync_copy(kif_hbm.at[pl.ds(base, PERB * L)], kf)

        def pr_step(i, _):
            o = i * L
            den = plsc.load_gather(acc, [kf[pl.ds(o, L)]]) + 1e-16
            exv[pl.ds(o, L)] = exv[pl.ds(o, L)] / den
            return 0

        lax.fori_loop(0, PERB, pr_step, 0)
        pltpu.sync_copy(exv, prob_hbm.at[pl.ds(cid * (Mp * L) + base,
                                               PERB * L)])

    return probs


def _make_upscatter(Mp, N1p, C):
    mesh = plsc.VectorSubcoreMesh(core_axis_name="c", subcore_axis_name="s")
    CPW = C // NW
    CPP = CPW // 2
    QCH = 256
    NCH = Mp // QCH

    @functools.partial(
        pl.kernel, mesh=mesh,
        compiler_params=pltpu.CompilerParams(needs_layout_passes=False),
        out_type=[
            jax.ShapeDtypeStruct((C * N1p,), jnp.float32),
        ],
        scratch_types=[
            pltpu.VMEM((CPP * N1p,), jnp.float32),
            pltpu.VMEM((QCH * L,), jnp.int32),
            pltpu.VMEM((QCH * L,), jnp.float32),
            pltpu.VMEM((CPP * Mp,), jnp.float32),
        ],
    )
    def upscatter(prob_hbm, kif_hbm, l2t_hbm, upt_hbm, buf, ck, cp, l2):
        cid = lax.axis_index("c")
        sid = lax.axis_index("s")
        wid = sid * NC + cid
        for p in range(2):
            gch = wid * CPW + p * CPP

            def ze_step(j, _):
                buf[pl.ds(j * L, L)] = jnp.zeros((L,), jnp.float32)
                return 0

            lax.fori_loop(0, (CPP * N1p) // L, ze_step, 0)
            pltpu.sync_copy(l2t_hbm.at[pl.ds(gch * Mp, CPP * Mp)], l2)

            def ch_step(c, _):
                cb = c * (QCH * L)
                pltpu.sync_copy(kif_hbm.at[pl.ds(cb, QCH * L)], ck)
                pltpu.sync_copy(
                    prob_hbm.at[pl.ds(cid * (Mp * L) + cb, QCH * L)], cp)

                def q_step(i, _):
                    o = i * L
                    kr = ck[pl.ds(o, L)]
                    pr = cp[pl.ds(o, L)]
                    qcol = jnp.full((L,), c * QCH + i, jnp.int32)
                    for ch in range(CPP):
                        val = plsc.load_gather(l2, [qcol + ch * Mp])
                        plsc.addupdate_scatter(
                            buf, [kr + ch * N1p], pr * val)
                    return 0

                lax.fori_loop(0, QCH, q_step, 0)
                return 0

            lax.fori_loop(0, NCH, ch_step, 0)
            for ch in range(CPP):
                pltpu.sync_copy(buf.at[pl.ds(ch * N1p, N1p)],
                                upt_hbm.at[pl.ds((gch + ch) * N1p, N1p)])

    return upscatter


def kernel(p1, x1, o1, p2, x2, o2, W1, b1, g1, be1, W2, b2, g2, be2,
           csW1, csb1, csg, csbe, csW2, csb2):
    n1 = p1.shape[0]
    m = p2.shape[0]
    C = x1.shape[1]
    Ci = x2.shape[1]

    PER = _rup(-(-m // NW), 8)
    Mp = PER * NW
    DSTP = _rup(_rup(n1, 2048) // NW, 8)
    N1p = DSTP * NW

    p1x = jnp.asarray(p1[:, 0], jnp.float32)
    p1y = jnp.asarray(p1[:, 1], jnp.float32)
    p1z = jnp.asarray(p1[:, 2], jnp.float32)
    p2Tp = jnp.pad(p2.T, ((0, 0), (0, Mp - m)))
    p2b = p2Tp.reshape(3, NW, PER).transpose(1, 0, 2).reshape(NW * 3 * PER)
    x2p = jnp.pad(x2, ((0, Mp - m), (0, 0)))
    V3p = jnp.pad(csW1[:3], ((0, 5), (0, 0)))
    w2row = csW2.reshape(1, Ci)

    lin1 = _lin1_call(x1, W1, b1, g1, be1)
    lin2T, A = _dense2_call(x2p, W2, b2, g2, be2, csW1[3:], csb1)

    knn = _make_knn(n1, PER, Mp)
    kif1, prx1, pry1, prz1 = knn(p1x, p1y, p1z, p2b)
    prx = prx1.reshape(Mp, L)
    pry = pry1.reshape(Mp, L)
    prz = prz1.reshape(Mp, L)

    shrink = _shrink_call(A, prx, pry, prz, V3p, csg, csbe, w2row, csb2)

    shrink1 = shrink.reshape(Mp * L)
    partials, exs = _make_exden(n1, m, Mp, N1p)(shrink1, kif1)
    probs, = _make_probs(m, Mp, N1p)(partials, exs, kif1)
    upTf, = _make_upscatter(Mp, N1p, C)(probs, kif1, lin2T.reshape(C * Mp))
    upT = upTf.reshape(C, N1p)

    lin1p = jnp.pad(lin1, ((0, N1p - n1), (0, 0)))
    return _final_call(lin1p, upT)[:n1]


# speedup vs baseline: 6.0695x; 1.0589x over previous
import functools
import math

import jax
import jax.numpy as jnp
from jax import lax
from jax.experimental import pallas as pl
from jax.experimental.pallas import tpu as pltpu
from jax.experimental.pallas import tpu_sc as plsc

NC = 2
NS = 16
NW = NC * NS
L = 16
_S = 1.0 / math.sqrt(1.0 + 1e-5)


def _rup(x, mult):
    return (x + mult - 1) // mult * mult


def _lin_body(x_ref, w_ref, b_ref, g_ref, be_ref, o_ref):
    acc = jnp.dot(x_ref[...], w_ref[...], preferred_element_type=jnp.float32)
    o_ref[...] = jnp.maximum(
        (acc + b_ref[...]) * (g_ref[...] * _S) + be_ref[...], 0.0)


def _lin1_call(x1, W1, b1, g1, be1):
    n, K = x1.shape
    C = W1.shape[1]
    RB = 2000
    return pl.pallas_call(
        _lin_body,
        grid=(n // RB,),
        in_specs=[
            pl.BlockSpec((RB, K), lambda i: (i, 0)),
            pl.BlockSpec((K, C), lambda i: (0, 0)),
            pl.BlockSpec((1, C), lambda i: (0, 0)),
            pl.BlockSpec((1, C), lambda i: (0, 0)),
            pl.BlockSpec((1, C), lambda i: (0, 0)),
        ],
        out_specs=pl.BlockSpec((RB, C), lambda i: (i, 0)),
        out_shape=jax.ShapeDtypeStruct((n, C), jnp.float32),
    )(x1, W1, b1.reshape(1, C), g1.reshape(1, C), be1.reshape(1, C))


def _dense2_body(x_ref, w2_ref, b2_ref, g2_ref, be2_ref, wc_ref, bc_ref,
                 l2t_ref, a_ref):
    x = x_ref[...]
    acc = jnp.dot(x, w2_ref[...], preferred_element_type=jnp.float32)
    lin2 = jnp.maximum(
        (acc + b2_ref[...]) * (g2_ref[...] * _S) + be2_ref[...], 0.0)
    l2t_ref[...] = lin2.T
    a_ref[...] = jnp.dot(x, wc_ref[...],
                         preferred_element_type=jnp.float32) + bc_ref[...]


def _dense2_call(x2p, W2, b2, g2, be2, Wc, bc):
    Mp, K = x2p.shape
    C = W2.shape[1]
    Ca = Wc.shape[1]
    RB = 1024
    return pl.pallas_call(
        _dense2_body,
        grid=(Mp // RB,),
        in_specs=[
            pl.BlockSpec((RB, K), lambda i: (i, 0)),
            pl.BlockSpec((K, C), lambda i: (0, 0)),
            pl.BlockSpec((1, C), lambda i: (0, 0)),
            pl.BlockSpec((1, C), lambda i: (0, 0)),
            pl.BlockSpec((1, C), lambda i: (0, 0)),
            pl.BlockSpec((K, Ca), lambda i: (0, 0)),
            pl.BlockSpec((1, Ca), lambda i: (0, 0)),
        ],
        out_specs=[
            pl.BlockSpec((C, RB), lambda i: (0, i)),
            pl.BlockSpec((RB, Ca), lambda i: (i, 0)),
        ],
        out_shape=[
            jax.ShapeDtypeStruct((C, Mp), jnp.float32),
            jax.ShapeDtypeStruct((Mp, Ca), jnp.float32),
        ],
    )(x2p, W2, b2.reshape(1, C), g2.reshape(1, C), be2.reshape(1, C),
      Wc, bc.reshape(1, Ca))


def _shrink_body(a_ref, px_ref, py_ref, pz_ref, v_ref, g_ref, be_ref, w2_ref,
                 b2_ref, o_ref):
    sg = g_ref[...] * _S
    a2 = a_ref[...] * sg + be_ref[...]
    v0 = v_ref[0:1, :] * sg
    v1 = v_ref[1:2, :] * sg
    v2 = v_ref[2:3, :] * sg
    w2 = w2_ref[...]
    cols = []
    for kk in range(L):
        b = (px_ref[:, kk:kk + 1] * v0 + py_ref[:, kk:kk + 1] * v1
             + pz_ref[:, kk:kk + 1] * v2)
        h = jnp.maximum(a2 + b, 0.0)
        cols.append(jnp.sum(h * w2, axis=1, keepdims=True))
    o_ref[...] = jnp.concatenate(cols, axis=1) + b2_ref[0, 0]


def _shrink_call(A, prx, pry, prz, V3p, csg, csbe, w2row, csb2):
    Mp, C = A.shape
    RB = 1024
    return pl.pallas_call(
        _shrink_body,
        grid=(Mp // RB,),
        in_specs=[
            pl.BlockSpec((RB, C), lambda i: (i, 0)),
            pl.BlockSpec((RB, L), lambda i: (i, 0)),
            pl.BlockSpec((RB, L), lambda i: (i, 0)),
            pl.BlockSpec((RB, L), lambda i: (i, 0)),
            pl.BlockSpec((8, C), lambda i: (0, 0)),
            pl.BlockSpec((1, C), lambda i: (0, 0)),
            pl.BlockSpec((1, C), lambda i: (0, 0)),
            pl.BlockSpec((1, C), lambda i: (0, 0)),
            pl.BlockSpec((1, 1), lambda i: (0, 0)),
        ],
        out_specs=pl.BlockSpec((RB, L), lambda i: (i, 0)),
        out_shape=jax.ShapeDtypeStruct((Mp, L), jnp.float32),
    )(A, prx, pry, prz, V3p, csg.reshape(1, C), csbe.reshape(1, C),
      w2row, csb2.reshape(1, 1))


def _final_body(l1_ref, upt_ref, o_ref):
    o_ref[...] = l1_ref[...] + upt_ref[...].T


def _final_call(lin1, upT):
    n, C = lin1.shape
    RB = 2048
    return pl.pallas_call(
        _final_body,
        grid=(n // RB,),
        in_specs=[
            pl.BlockSpec((RB, C), lambda i: (i, 0)),
            pl.BlockSpec((C, RB), lambda i: (0, i)),
        ],
        out_specs=pl.BlockSpec((RB, C), lambda i: (i, 0)),
        out_shape=jax.ShapeDtypeStruct((n, C), jnp.float32),
    )(lin1, upT)


def _bf16r(x):
    u = plsc.bitcast(x, jnp.int32)
    b = 32767 + ((u >> 16) & 1)
    return plsc.bitcast((u + b) & (-65536), jnp.float32)


def _make_knn(n1, PER, Mp):
    mesh = plsc.VectorSubcoreMesh(core_axis_name="c", subcore_axis_name="s")
    NSTEP = n1 // L
    R = 6
    RFB = 16
    LOWM = 0x7FF

    @functools.partial(
        pl.kernel, mesh=mesh,
        compiler_params=pltpu.CompilerParams(needs_layout_passes=False),
        out_type=[
            jax.ShapeDtypeStruct((Mp * L,), jnp.int32),
            jax.ShapeDtypeStruct((Mp * L,), jnp.float32),
            jax.ShapeDtypeStruct((Mp * L,), jnp.float32),
            jax.ShapeDtypeStruct((Mp * L,), jnp.float32),
        ],
        scratch_types=[
            pltpu.VMEM((n1,), jnp.float32),
            pltpu.VMEM((n1,), jnp.float32),
            pltpu.VMEM((n1,), jnp.float32),
            pltpu.VMEM((n1,), jnp.float32),
            pltpu.VMEM((3 * PER,), jnp.float32),
            pltpu.VMEM((PER * L,), jnp.int32),
            pltpu.VMEM((PER * L,), jnp.float32),
            pltpu.VMEM((PER * L,), jnp.float32),
            pltpu.VMEM((PER * L,), jnp.float32),
        ],
    )
    def knn(p1x_hbm, p1y_hbm, p1z_hbm, p2b_hbm, oidx, oprx, opry, oprz,
            p1a, p1b, p1c, p1q, qv, bidx, bx, by, bz):
        wid = lax.axis_index("s") * NC + lax.axis_index("c")
        pltpu.sync_copy(p1x_hbm, p1a)
        pltpu.sync_copy(p1y_hbm, p1b)
        pltpu.sync_copy(p1z_hbm, p1c)
        pltpu.sync_copy(p2b_hbm.at[pl.ds(wid * 3 * PER, 3 * PER)], qv)

        def sq_step(j, _):
            o = j * L
            px = p1a[pl.ds(o, L)]
            py = p1b[pl.ds(o, L)]
            pz = p1c[pl.ds(o, L)]
            p1q[pl.ds(o, L)] = px * px + py * py + pz * pz
            p1a[pl.ds(o, L)] = -2.0 * _bf16r(px)
            p1b[pl.ds(o, L)] = -2.0 * _bf16r(py)
            p1c[pl.ds(o, L)] = -2.0 * _bf16r(pz)
            return 0

        lax.fori_loop(0, NSTEP, sq_step, 0)

        iota = lax.iota(jnp.int32, L)
        inf16 = jnp.full((L,), jnp.inf, jnp.float32)

        def per_query(i, _):
            iv = jnp.full((L,), i, jnp.int32)
            qx = plsc.load_gather(qv, [iv])
            qy = plsc.load_gather(qv, [iv + PER])
            qz = plsc.load_gather(qv, [iv + 2 * PER])
            qsq = qx * qx + qy * qy + qz * qz
            qxb = _bf16r(qx)
            qyb = _bf16r(qy)
            qzb = _bf16r(qz)

            def dvec(s):
                o = s * L
                return ((p1q[pl.ds(o, L)] + qsq)
                        + (qxb * p1a[pl.ds(o, L)] + qyb * p1b[pl.ds(o, L)]
                           + qzb * p1c[pl.ds(o, L)]))

            def step_fn(s, A):
                u = plsc.bitcast(dvec(s), jnp.int32)
                t = plsc.bitcast((u & (~LOWM)) | s, jnp.float32)
                out = []
                for r in range(R):
                    out.append(jnp.minimum(A[r], t))
                    t = jnp.maximum(A[r], t)
                return tuple(out)

            A = lax.fori_loop(0, NSTEP, step_fn, (inf16,) * R)

            M_d = None
            M_i = None
            for r in range(R):
                u = plsc.bitcast(A[r], jnp.int32)
                ci = (u & LOWM) * L + iota
                de = ((plsc.load_gather(p1q, [ci]) + qsq)
                      + (qxb * plsc.load_gather(p1a, [ci])
                         + qyb * plsc.load_gather(p1b, [ci])
                         + qzb * plsc.load_gather(p1c, [ci])))
                Cs, Ci = plsc.sort_key_val(de, ci)
                if r == 0:
                    M_d, M_i = Cs, Ci
                else:
                    rC = lax.rev(Cs, (0,))
                    rCi = lax.rev(Ci, (0,))
                    mm = M_d <= rC
                    nd = jnp.where(mm, M_d, rC)
                    ni = jnp.where(mm, M_i, rCi)
                    M_d, M_i = plsc.sort_key_val(nd, ni)

            tau = jnp.broadcast_to(jnp.max(M_d), (L,))
            cl = plsc.bitcast(plsc.bitcast(A[R - 1], jnp.int32) & (~LOWM),
                              jnp.float32)
            bad = jnp.any(cl - 0.002 * jnp.abs(cl) < tau)

            def exact(_):
                def step2(s, A2):
                    u = plsc.bitcast(dvec(s), jnp.int32)
                    t = plsc.bitcast((u & (~LOWM)) | s, jnp.float32)
                    out = []
                    for r in range(RFB):
                        out.append(jnp.minimum(A2[r], t))
                        t = jnp.maximum(A2[r], t)
                    return tuple(out)

                A2 = lax.fori_loop(0, NSTEP, step2, (inf16,) * RFB)
                Fd = None
                Fi = None
                for r in range(RFB):
                    u = plsc.bitcast(A2[r], jnp.int32)
                    ci = (u & LOWM) * L + iota
                    de = ((plsc.load_gather(p1q, [ci]) + qsq)
                          + (qxb * plsc.load_gather(p1a, [ci])
                             + qyb * plsc.load_gather(p1b, [ci])
                             + qzb * plsc.load_gather(p1c, [ci])))
                    Cs, Ci = plsc.sort_key_val(de, ci)
                    if r == 0:
                        Fd, Fi = Cs, Ci
                    else:
                        rC = lax.rev(Cs, (0,))
                        rCi = lax.rev(Ci, (0,))
                        mm = Fd <= rC
                        nd = jnp.where(mm, Fd, rC)
                        ni = jnp.where(mm, Fi, rCi)
                        Fd, Fi = plsc.sort_key_val(nd, ni)
                return Fd, Fi

            M_d, M_i = lax.cond(bad, exact, lambda _: (M_d, M_i), 0)
            bidx[pl.ds(i * L, L)] = M_i
            return 0

        lax.fori_loop(0, PER, per_query, 0)

        pltpu.sync_copy(p1x_hbm, p1a)
        pltpu.sync_copy(p1y_hbm, p1b)
        pltpu.sync_copy(p1z_hbm, p1c)

        def pr_query(i, _):
            iv = jnp.full((L,), i, jnp.int32)
            qx = plsc.load_gather(qv, [iv])
            qy = plsc.load_gather(qv, [iv + PER])
            qz = plsc.load_gather(qv, [iv + 2 * PER])
            o = i * L
            idx = bidx[pl.ds(o, L)]
            bx[pl.ds(o, L)] = plsc.load_gather(p1a, [idx]) - qx
            by[pl.ds(o, L)] = plsc.load_gather(p1b, [idx]) - qy
            bz[pl.ds(o, L)] = plsc.load_gather(p1c, [idx]) - qz
            return 0

        lax.fori_loop(0, PER, pr_query, 0)

        base = wid * (PER * L)
        pltpu.sync_copy(bidx, oidx.at[pl.ds(base, PER * L)])
        pltpu.sync_copy(bx, oprx.at[pl.ds(base, PER * L)])
        pltpu.sync_copy(by, opry.at[pl.ds(base, PER * L)])
        pltpu.sync_copy(bz, oprz.at[pl.ds(base, PER * L)])

    return knn


def _make_exden(n1, m, Mp, N1p):
    mesh = plsc.VectorSubcoreMesh(core_axis_name="c", subcore_axis_name="s")
    PERB = Mp // NS

    @functools.partial(
        pl.kernel, mesh=mesh,
        compiler_params=pltpu.CompilerParams(needs_layout_passes=False),
        out_type=[
            jax.ShapeDtypeStruct((NC * NS * N1p,), jnp.float32),
            jax.ShapeDtypeStruct((NC * Mp * L,), jnp.float32),
        ],
        scratch_types=[
            pltpu.VMEM((PERB * L,), jnp.float32),
            pltpu.VMEM((PERB * L,), jnp.int32),
            pltpu.VMEM((N1p,), jnp.float32),
        ],
    )
    def exden(shrink_hbm, kif_hbm, part_hbm, ex_hbm, sh, kf, buf):
        cid = lax.axis_index("c")
        sid = lax.axis_index("s")

        def mx_chunk(c, mv):
            pltpu.sync_copy(shrink_hbm.at[pl.ds(c * (PERB * L), PERB * L)], sh)

            def mx_step(i, mv):
                return jnp.maximum(mv, sh[pl.ds(i * L, L)])

            return lax.fori_loop(0, PERB, mx_step, mv)

        mv = lax.fori_loop(0, NS, mx_chunk,
                           jnp.full((L,), -jnp.inf, jnp.float32))
        gmax = jnp.broadcast_to(jnp.max(mv), (L,))

        base = sid * (PERB * L)
        pltpu.sync_copy(shrink_hbm.at[pl.ds(base, PERB * L)], sh)
        pltpu.sync_copy(kif_hbm.at[pl.ds(base, PERB * L)], kf)

        def z_step(j, _):
            buf[pl.ds(j * L, L)] = jnp.zeros((L,), jnp.float32)
            return 0

        lax.fori_loop(0, N1p // L, z_step, 0)
        start_q = sid * PERB

        def ex_step(i, _):
            o = i * L
            e = jnp.exp(sh[pl.ds(o, L)] - gmax)
            e = e * ((start_q + i) < m).astype(jnp.float32)
            sh[pl.ds(o, L)] = e
            plsc.addupdate_scatter(buf, [kf[pl.ds(o, L)]], e)
            return 0

        lax.fori_loop(0, PERB, ex_step, 0)
        pltpu.sync_copy(buf, part_hbm.at[pl.ds((cid * NS + sid) * N1p, N1p)])
        pltpu.sync_copy(sh, ex_hbm.at[pl.ds(cid * (Mp * L) + base, PERB * L)])

    return exden


def _make_probs(m, Mp, N1p):
    mesh = plsc.VectorSubcoreMesh(core_axis_name="c", subcore_axis_name="s")
    PERB = Mp // NS

    @functools.partial(
        pl.kernel, mesh=mesh,
        compiler_params=pltpu.CompilerParams(needs_layout_passes=False),
        out_type=[
            jax.ShapeDtypeStruct((NC * Mp * L,), jnp.float32),
        ],
        scratch_types=[
            pltpu.VMEM((N1p,), jnp.float32),
            pltpu.VMEM((N1p,), jnp.float32),
            pltpu.VMEM((PERB * L,), jnp.float32),
            pltpu.VMEM((PERB * L,), jnp.int32),
        ],
    )
    def probs(part_hbm, ex_hbm, kif_hbm, prob_hbm, acc, tmp, exv, kf):
        cid = lax.axis_index("c")
        sid = lax.axis_index("s")
        pbase = cid * NS * N1p
        pltpu.sync_copy(part_hbm.at[pl.ds(pbase, N1p)], acc)

        def red_step(c, _):
            pltpu.sync_copy(part_hbm.at[pl.ds(pbase + c * N1p, N1p)], tmp)

            def add_step(j, _):
                o = j * (4 * L)
                for k in range(4):
                    ok = o + k * L
                    acc[pl.ds(ok, L)] = acc[pl.ds(ok, L)] + tmp[pl.ds(ok, L)]
                return 0

            lax.fori_loop(0, N1p // (4 * L), add_step, 0)
            return 0

        lax.fori_loop(1, NS, red_step, 0)

        base = sid * (PERB * L)
        pltpu.sync_copy(ex_hbm.at[pl.ds(cid * (Mp * L) + base, PERB * L)], exv)
        pltpu.sync_copy(kif_hbm.at[pl.ds(base, PERB * L)], kf)

        def pr_step(i, _):
            o = i * L
            den = plsc.load_gather(acc, [kf[pl.ds(o, L)]]) + 1e-16
            exv[pl.ds(o, L)] = exv[pl.ds(o, L)] / den
            return 0

        lax.fori_loop(0, PERB, pr_step, 0)
        pltpu.sync_copy(exv, prob_hbm.at[pl.ds(cid * (Mp * L) + base,
                                               PERB * L)])

    return probs


def _make_upscatter(Mp, N1p, C):
    mesh = plsc.VectorSubcoreMesh(core_axis_name="c", subcore_axis_name="s")
    CPW = C // NW
    CPP = CPW // 2
    QCH = 512
    NCH = Mp // QCH

    @functools.partial(
        pl.kernel, mesh=mesh,
        compiler_params=pltpu.CompilerParams(needs_layout_passes=False),
        out_type=[
            jax.ShapeDtypeStruct((C * N1p,), jnp.float32),
        ],
        scratch_types=[
            pltpu.VMEM((CPP * N1p,), jnp.float32),
            pltpu.VMEM((QCH * L,), jnp.int32),
            pltpu.VMEM((QCH * L,), jnp.float32),
            pltpu.VMEM((CPP * Mp,), jnp.float32),
        ],
    )
    def upscatter(prob_hbm, kif_hbm, l2t_hbm, upt_hbm, buf, ck, cp, l2):
        cid = lax.axis_index("c")
        sid = lax.axis_index("s")
        wid = sid * NC + cid
        for p in range(2):
            gch = wid * CPW + p * CPP

            def ze_step(j, _):
                buf[pl.ds(j * L, L)] = jnp.zeros((L,), jnp.float32)
                return 0

            lax.fori_loop(0, (CPP * N1p) // L, ze_step, 0)
            pltpu.sync_copy(l2t_hbm.at[pl.ds(gch * Mp, CPP * Mp)], l2)

            def ch_step(c, _):
                cb = c * (QCH * L)
                pltpu.sync_copy(kif_hbm.at[pl.ds(cb, QCH * L)], ck)
                pltpu.sync_copy(
                    prob_hbm.at[pl.ds(cid * (Mp * L) + cb, QCH * L)], cp)

                def q_step(i, _):
                    for half in range(2):
                        ii = i * 2 + half
                        o = ii * L
                        kr = ck[pl.ds(o, L)]
                        pr = cp[pl.ds(o, L)]
                        qcol = jnp.full((L,), c * QCH + ii, jnp.int32)
                        for ch in range(CPP):
                            val = plsc.load_gather(l2, [qcol + ch * Mp])
                            plsc.addupdate_scatter(
                                buf, [kr + ch * N1p], pr * val)
                    return 0

                lax.fori_loop(0, QCH // 2, q_step, 0)
                return 0

            lax.fori_loop(0, NCH, ch_step, 0)
            for ch in range(CPP):
                pltpu.sync_copy(buf.at[pl.ds(ch * N1p, N1p)],
                                upt_hbm.at[pl.ds((gch + ch) * N1p, N1p)])

    return upscatter


def kernel(p1, x1, o1, p2, x2, o2, W1, b1, g1, be1, W2, b2, g2, be2,
           csW1, csb1, csg, csbe, csW2, csb2):
    n1 = p1.shape[0]
    m = p2.shape[0]
    C = x1.shape[1]
    Ci = x2.shape[1]

    PER = _rup(-(-m // NW), 8)
    Mp = PER * NW
    DSTP = _rup(_rup(n1, 2048) // NW, 8)
    N1p = DSTP * NW

    p1x = jnp.asarray(p1[:, 0], jnp.float32)
    p1y = jnp.asarray(p1[:, 1], jnp.float32)
    p1z = jnp.asarray(p1[:, 2], jnp.float32)
    p2Tp = jnp.pad(p2.T, ((0, 0), (0, Mp - m)))
    p2b = p2Tp.reshape(3, NW, PER).transpose(1, 0, 2).reshape(NW * 3 * PER)
    x2p = jnp.pad(x2, ((0, Mp - m), (0, 0)))
    V3p = jnp.pad(csW1[:3], ((0, 5), (0, 0)))
    w2row = csW2.reshape(1, Ci)

    lin1 = _lin1_call(x1, W1, b1, g1, be1)
    lin2T, A = _dense2_call(x2p, W2, b2, g2, be2, csW1[3:], csb1)

    knn = _make_knn(n1, PER, Mp)
    kif1, prx1, pry1, prz1 = knn(p1x, p1y, p1z, p2b)
    prx = prx1.reshape(Mp, L)
    pry = pry1.reshape(Mp, L)
    prz = prz1.reshape(Mp, L)

    shrink = _shrink_call(A, prx, pry, prz, V3p, csg, csbe, w2row, csb2)

    shrink1 = shrink.reshape(Mp * L)
    partials, exs = _make_exden(n1, m, Mp, N1p)(shrink1, kif1)
    probs, = _make_probs(m, Mp, N1p)(partials, exs, kif1)
    upTf, = _make_upscatter(Mp, N1p, C)(probs, kif1, lin2T.reshape(C * Mp))
    upT = upTf.reshape(C, N1p)

    lin1p = jnp.pad(lin1, ((0, N1p - n1), (0, 0)))
    return _final_call(lin1p, upT)[:n1]


# speedup vs baseline: 6.2340x; 1.0271x over previous
import functools
import math

import jax
import jax.numpy as jnp
from jax import lax
from jax.experimental import pallas as pl
from jax.experimental.pallas import tpu as pltpu
from jax.experimental.pallas import tpu_sc as plsc

NC = 2
NS = 16
NW = NC * NS
L = 16
_S = 1.0 / math.sqrt(1.0 + 1e-5)


def _rup(x, mult):
    return (x + mult - 1) // mult * mult


def _lin_body(x_ref, w_ref, b_ref, g_ref, be_ref, o_ref):
    acc = jnp.dot(x_ref[...], w_ref[...], preferred_element_type=jnp.float32)
    o_ref[...] = jnp.maximum(
        (acc + b_ref[...]) * (g_ref[...] * _S) + be_ref[...], 0.0)


def _lin1_call(x1, W1, b1, g1, be1):
    n, K = x1.shape
    C = W1.shape[1]
    RB = 2000
    return pl.pallas_call(
        _lin_body,
        grid=(n // RB,),
        in_specs=[
            pl.BlockSpec((RB, K), lambda i: (i, 0)),
            pl.BlockSpec((K, C), lambda i: (0, 0)),
            pl.BlockSpec((1, C), lambda i: (0, 0)),
            pl.BlockSpec((1, C), lambda i: (0, 0)),
            pl.BlockSpec((1, C), lambda i: (0, 0)),
        ],
        out_specs=pl.BlockSpec((RB, C), lambda i: (i, 0)),
        out_shape=jax.ShapeDtypeStruct((n, C), jnp.float32),
    )(x1, W1, b1.reshape(1, C), g1.reshape(1, C), be1.reshape(1, C))


def _dense2_body(x_ref, w2_ref, b2_ref, g2_ref, be2_ref, wc_ref, bc_ref,
                 l2t_ref, a_ref):
    x = x_ref[...]
    acc = jnp.dot(x, w2_ref[...], preferred_element_type=jnp.float32)
    lin2 = jnp.maximum(
        (acc + b2_ref[...]) * (g2_ref[...] * _S) + be2_ref[...], 0.0)
    l2t_ref[...] = lin2.T
    a_ref[...] = jnp.dot(x, wc_ref[...],
                         preferred_element_type=jnp.float32) + bc_ref[...]


def _dense2_call(x2p, W2, b2, g2, be2, Wc, bc):
    Mp, K = x2p.shape
    C = W2.shape[1]
    Ca = Wc.shape[1]
    RB = 1024
    return pl.pallas_call(
        _dense2_body,
        grid=(Mp // RB,),
        in_specs=[
            pl.BlockSpec((RB, K), lambda i: (i, 0)),
            pl.BlockSpec((K, C), lambda i: (0, 0)),
            pl.BlockSpec((1, C), lambda i: (0, 0)),
            pl.BlockSpec((1, C), lambda i: (0, 0)),
            pl.BlockSpec((1, C), lambda i: (0, 0)),
            pl.BlockSpec((K, Ca), lambda i: (0, 0)),
            pl.BlockSpec((1, Ca), lambda i: (0, 0)),
        ],
        out_specs=[
            pl.BlockSpec((C, RB), lambda i: (0, i)),
            pl.BlockSpec((RB, Ca), lambda i: (i, 0)),
        ],
        out_shape=[
            jax.ShapeDtypeStruct((C, Mp), jnp.float32),
            jax.ShapeDtypeStruct((Mp, Ca), jnp.float32),
        ],
    )(x2p, W2, b2.reshape(1, C), g2.reshape(1, C), be2.reshape(1, C),
      Wc, bc.reshape(1, Ca))


def _shrink_body(a_ref, px_ref, py_ref, pz_ref, v_ref, g_ref, be_ref, w2_ref,
                 b2_ref, o_ref):
    sg = g_ref[...] * _S
    a2 = a_ref[...] * sg + be_ref[...]
    v0 = v_ref[0:1, :] * sg
    v1 = v_ref[1:2, :] * sg
    v2 = v_ref[2:3, :] * sg
    w2 = w2_ref[...]
    cols = []
    for kk in range(L):
        b = (px_ref[:, kk:kk + 1] * v0 + py_ref[:, kk:kk + 1] * v1
             + pz_ref[:, kk:kk + 1] * v2)
        h = jnp.maximum(a2 + b, 0.0)
        cols.append(jnp.sum(h * w2, axis=1, keepdims=True))
    o_ref[...] = jnp.concatenate(cols, axis=1) + b2_ref[0, 0]


def _shrink_call(A, prx, pry, prz, V3p, csg, csbe, w2row, csb2):
    Mp, C = A.shape
    RB = 1024
    return pl.pallas_call(
        _shrink_body,
        grid=(Mp // RB,),
        in_specs=[
            pl.BlockSpec((RB, C), lambda i: (i, 0)),
            pl.BlockSpec((RB, L), lambda i: (i, 0)),
            pl.BlockSpec((RB, L), lambda i: (i, 0)),
            pl.BlockSpec((RB, L), lambda i: (i, 0)),
            pl.BlockSpec((8, C), lambda i: (0, 0)),
            pl.BlockSpec((1, C), lambda i: (0, 0)),
            pl.BlockSpec((1, C), lambda i: (0, 0)),
            pl.BlockSpec((1, C), lambda i: (0, 0)),
            pl.BlockSpec((1, 1), lambda i: (0, 0)),
        ],
        out_specs=pl.BlockSpec((RB, L), lambda i: (i, 0)),
        out_shape=jax.ShapeDtypeStruct((Mp, L), jnp.float32),
    )(A, prx, pry, prz, V3p, csg.reshape(1, C), csbe.reshape(1, C),
      w2row, csb2.reshape(1, 1))


def _final_body(l1_ref, upt_ref, o_ref):
    o_ref[...] = l1_ref[...] + upt_ref[...].T


def _final_call(lin1, upT):
    n, C = lin1.shape
    RB = 2048
    return pl.pallas_call(
        _final_body,
        grid=(n // RB,),
        in_specs=[
            pl.BlockSpec((RB, C), lambda i: (i, 0)),
            pl.BlockSpec((C, RB), lambda i: (0, i)),
        ],
        out_specs=pl.BlockSpec((RB, C), lambda i: (i, 0)),
        out_shape=jax.ShapeDtypeStruct((n, C), jnp.float32),
    )(lin1, upT)


def _bf16r(x):
    u = plsc.bitcast(x, jnp.int32)
    b = 32767 + ((u >> 16) & 1)
    return plsc.bitcast((u + b) & (-65536), jnp.float32)


def _make_knn(n1, PER, Mp):
    mesh = plsc.VectorSubcoreMesh(core_axis_name="c", subcore_axis_name="s")
    NSTEP = n1 // L
    R = 5
    RFB = 16
    LOWM = 0x7FF

    @functools.partial(
        pl.kernel, mesh=mesh,
        compiler_params=pltpu.CompilerParams(needs_layout_passes=False),
        out_type=[
            jax.ShapeDtypeStruct((Mp * L,), jnp.int32),
            jax.ShapeDtypeStruct((Mp * L,), jnp.float32),
            jax.ShapeDtypeStruct((Mp * L,), jnp.float32),
            jax.ShapeDtypeStruct((Mp * L,), jnp.float32),
        ],
        scratch_types=[
            pltpu.VMEM((n1,), jnp.float32),
            pltpu.VMEM((n1,), jnp.float32),
            pltpu.VMEM((n1,), jnp.float32),
            pltpu.VMEM((n1,), jnp.float32),
            pltpu.VMEM((3 * PER,), jnp.float32),
            pltpu.VMEM((PER * L,), jnp.int32),
            pltpu.VMEM((PER * L,), jnp.float32),
            pltpu.VMEM((PER * L,), jnp.float32),
            pltpu.VMEM((PER * L,), jnp.float32),
        ],
    )
    def knn(p1x_hbm, p1y_hbm, p1z_hbm, p2b_hbm, oidx, oprx, opry, oprz,
            p1a, p1b, p1c, p1q, qv, bidx, bx, by, bz):
        wid = lax.axis_index("s") * NC + lax.axis_index("c")
        pltpu.sync_copy(p1x_hbm, p1a)
        pltpu.sync_copy(p1y_hbm, p1b)
        pltpu.sync_copy(p1z_hbm, p1c)
        pltpu.sync_copy(p2b_hbm.at[pl.ds(wid * 3 * PER, 3 * PER)], qv)

        def sq_step(j, _):
            o = j * L
            px = p1a[pl.ds(o, L)]
            py = p1b[pl.ds(o, L)]
            pz = p1c[pl.ds(o, L)]
            p1q[pl.ds(o, L)] = px * px + py * py + pz * pz
            p1a[pl.ds(o, L)] = -2.0 * _bf16r(px)
            p1b[pl.ds(o, L)] = -2.0 * _bf16r(py)
            p1c[pl.ds(o, L)] = -2.0 * _bf16r(pz)
            return 0

        lax.fori_loop(0, NSTEP, sq_step, 0)

        iota = lax.iota(jnp.int32, L)
        inf16 = jnp.full((L,), jnp.inf, jnp.float32)

        def per_query(i, _):
            iv = jnp.full((L,), i, jnp.int32)
            qx = plsc.load_gather(qv, [iv])
            qy = plsc.load_gather(qv, [iv + PER])
            qz = plsc.load_gather(qv, [iv + 2 * PER])
            qsq = qx * qx + qy * qy + qz * qz
            qxb = _bf16r(qx)
            qyb = _bf16r(qy)
            qzb = _bf16r(qz)

            def dvec(s):
                o = s * L
                return ((p1q[pl.ds(o, L)] + qsq)
                        + (qxb * p1a[pl.ds(o, L)] + qyb * p1b[pl.ds(o, L)]
                           + qzb * p1c[pl.ds(o, L)]))

            def step_fn(s, A):
                u = plsc.bitcast(dvec(s), jnp.int32)
                t = plsc.bitcast((u & (~LOWM)) | s, jnp.float32)
                out = []
                for r in range(R):
                    out.append(jnp.minimum(A[r], t))
                    t = jnp.maximum(A[r], t)
                return tuple(out)

            A = lax.fori_loop(0, NSTEP, step_fn, (inf16,) * R)

            M_d = None
            M_i = None
            for r in range(R):
                u = plsc.bitcast(A[r], jnp.int32)
                ci = (u & LOWM) * L + iota
                de = ((plsc.load_gather(p1q, [ci]) + qsq)
                      + (qxb * plsc.load_gather(p1a, [ci])
                         + qyb * plsc.load_gather(p1b, [ci])
                         + qzb * plsc.load_gather(p1c, [ci])))
                Cs, Ci = plsc.sort_key_val(de, ci)
                if r == 0:
                    M_d, M_i = Cs, Ci
                else:
                    rC = lax.rev(Cs, (0,))
                    rCi = lax.rev(Ci, (0,))
                    mm = M_d <= rC
                    nd = jnp.where(mm, M_d, rC)
                    ni = jnp.where(mm, M_i, rCi)
                    M_d, M_i = plsc.sort_key_val(nd, ni)

            tau = jnp.broadcast_to(jnp.max(M_d), (L,))
            cl = plsc.bitcast(plsc.bitcast(A[R - 1], jnp.int32) & (~LOWM),
                              jnp.float32)
            bad = jnp.any(cl - 0.002 * jnp.abs(cl) < tau)

            def exact(_):
                def step2(s, A2):
                    u = plsc.bitcast(dvec(s), jnp.int32)
                    t = plsc.bitcast((u & (~LOWM)) | s, jnp.float32)
                    out = []
                    for r in range(RFB):
                        out.append(jnp.minimum(A2[r], t))
                        t = jnp.maximum(A2[r], t)
                    return tuple(out)

                A2 = lax.fori_loop(0, NSTEP, step2, (inf16,) * RFB)
                Fd = None
                Fi = None
                for r in range(RFB):
                    u = plsc.bitcast(A2[r], jnp.int32)
                    ci = (u & LOWM) * L + iota
                    de = ((plsc.load_gather(p1q, [ci]) + qsq)
                          + (qxb * plsc.load_gather(p1a, [ci])
                             + qyb * plsc.load_gather(p1b, [ci])
                             + qzb * plsc.load_gather(p1c, [ci])))
                    Cs, Ci = plsc.sort_key_val(de, ci)
                    if r == 0:
                        Fd, Fi = Cs, Ci
                    else:
                        rC = lax.rev(Cs, (0,))
                        rCi = lax.rev(Ci, (0,))
                        mm = Fd <= rC
                        nd = jnp.where(mm, Fd, rC)
                        ni = jnp.where(mm, Fi, rCi)
                        Fd, Fi = plsc.sort_key_val(nd, ni)
                return Fd, Fi

            M_d, M_i = lax.cond(bad, exact, lambda _: (M_d, M_i), 0)
            bidx[pl.ds(i * L, L)] = M_i
            return 0

        lax.fori_loop(0, PER, per_query, 0)

        pltpu.sync_copy(p1x_hbm, p1a)
        pltpu.sync_copy(p1y_hbm, p1b)
        pltpu.sync_copy(p1z_hbm, p1c)

        def pr_query(i, _):
            iv = jnp.full((L,), i, jnp.int32)
            qx = plsc.load_gather(qv, [iv])
            qy = plsc.load_gather(qv, [iv + PER])
            qz = plsc.load_gather(qv, [iv + 2 * PER])
            o = i * L
            idx = bidx[pl.ds(o, L)]
            bx[pl.ds(o, L)] = plsc.load_gather(p1a, [idx]) - qx
            by[pl.ds(o, L)] = plsc.load_gather(p1b, [idx]) - qy
            bz[pl.ds(o, L)] = plsc.load_gather(p1c, [idx]) - qz
            return 0

        lax.fori_loop(0, PER, pr_query, 0)

        base = wid * (PER * L)
        pltpu.sync_copy(bidx, oidx.at[pl.ds(base, PER * L)])
        pltpu.sync_copy(bx, oprx.at[pl.ds(base, PER * L)])
        pltpu.sync_copy(by, opry.at[pl.ds(base, PER * L)])
        pltpu.sync_copy(bz, oprz.at[pl.ds(base, PER * L)])

    return knn


def _make_exden(n1, m, Mp, N1p):
    mesh = plsc.VectorSubcoreMesh(core_axis_name="c", subcore_axis_name="s")
    PERB = Mp // NS

    @functools.partial(
        pl.kernel, mesh=mesh,
        compiler_params=pltpu.CompilerParams(needs_layout_passes=False),
        out_type=[
            jax.ShapeDtypeStruct((NC * NS * N1p,), jnp.float32),
            jax.ShapeDtypeStruct((NC * Mp * L,), jnp.float32),
        ],
        scratch_types=[
            pltpu.VMEM((PERB * L,), jnp.float32),
            pltpu.VMEM((PERB * L,), jnp.int32),
            pltpu.VMEM((N1p,), jnp.float32),
        ],
    )
    def exden(shrink_hbm, kif_hbm, part_hbm, ex_hbm, sh, kf, buf):
        cid = lax.axis_index("c")
        sid = lax.axis_index("s")

        def mx_chunk(c, mv):
            pltpu.sync_copy(shrink_hbm.at[pl.ds(c * (PERB * L), PERB * L)], sh)

            def mx_step(i, mv):
                return jnp.maximum(mv, sh[pl.ds(i * L, L)])

            return lax.fori_loop(0, PERB, mx_step, mv)

        mv = lax.fori_loop(0, NS, mx_chunk,
                           jnp.full((L,), -jnp.inf, jnp.float32))
        gmax = jnp.broadcast_to(jnp.max(mv), (L,))

        base = sid * (PERB * L)
        pltpu.sync_copy(shrink_hbm.at[pl.ds(base, PERB * L)], sh)
        pltpu.sync_copy(kif_hbm.at[pl.ds(base, PERB * L)], kf)

        def z_step(j, _):
            buf[pl.ds(j * L, L)] = jnp.zeros((L,), jnp.float32)
            return 0

        lax.fori_loop(0, N1p // L, z_step, 0)
        start_q = sid * PERB

        def ex_step(i, _):
            o = i * L
            e = jnp.exp(sh[pl.ds(o, L)] - gmax)
            e = e * ((start_q + i) < m).astype(jnp.float32)
            sh[pl.ds(o, L)] = e
            plsc.addupdate_scatter(buf, [kf[pl.ds(o, L)]], e)
            return 0

        lax.fori_loop(0, PERB, ex_step, 0)
        pltpu.sync_copy(buf, part_hbm.at[pl.ds((cid * NS + sid) * N1p, N1p)])
        pltpu.sync_copy(sh, ex_hbm.at[pl.ds(cid * (Mp * L) + base, PERB * L)])

    return exden


def _make_probs(m, Mp, N1p):
    mesh = plsc.VectorSubcoreMesh(core_axis_name="c", subcore_axis_name="s")
    PERB = Mp // NS

    @functools.partial(
        pl.kernel, mesh=mesh,
        compiler_params=pltpu.CompilerParams(needs_layout_passes=False),
        out_type=[
            jax.ShapeDtypeStruct((NC * Mp * L,), jnp.float32),
        ],
        scratch_types=[
            pltpu.VMEM((N1p,), jnp.float32),
            pltpu.VMEM((N1p,), jnp.float32),
            pltpu.VMEM((PERB * L,), jnp.float32),
            pltpu.VMEM((PERB * L,), jnp.int32),
        ],
    )
    def probs(part_hbm, ex_hbm, kif_hbm, prob_hbm, acc, tmp, exv, kf):
        cid = lax.axis_index("c")
        sid = lax.axis_index("s")
        pbase = cid * NS * N1p
        pltpu.sync_copy(part_hbm.at[pl.ds(pbase, N1p)], acc)

        def red_step(c, _):
            pltpu.sync_copy(part_hbm.at[pl.ds(pbase + c * N1p, N1p)], tmp)

            def add_step(j, _):
                o = j * (4 * L)
                for k in range(4):
                    ok = o + k * L
                    acc[pl.ds(ok, L)] = acc[pl.ds(ok, L)] + tmp[pl.ds(ok, L)]
                return 0

            lax.fori_loop(0, N1p // (4 * L), add_step, 0)
            return 0

        lax.fori_loop(1, NS, red_step, 0)

        base = sid * (PERB * L)
        pltpu.sync_copy(ex_hbm.at[pl.ds(cid * (Mp * L) + base, PERB * L)], exv)
        pltpu.sync_copy(kif_hbm.at[pl.ds(base, PERB * L)], kf)

        def pr_step(i, _):
            o = i * L
            den = plsc.load_gather(acc, [kf[pl.ds(o, L)]]) + 1e-16
            exv[pl.ds(o, L)] = exv[pl.ds(o, L)] / den
            return 0

        lax.fori_loop(0, PERB, pr_step, 0)
        pltpu.sync_copy(exv, prob_hbm.at[pl.ds(cid * (Mp * L) + base,
                                               PERB * L)])

    return probs


def _make_upscatter(Mp, N1p, C):
    mesh = plsc.VectorSubcoreMesh(core_axis_name="c", subcore_axis_name="s")
    CPW = C // NW
    CPP = CPW // 2
    QCH = 512
    NCH = Mp // QCH

    @functools.partial(
        pl.kernel, mesh=mesh,
        compiler_params=pltpu.CompilerParams(needs_layout_passes=False),
        out_type=[
            jax.ShapeDtypeStruct((C * N1p,), jnp.float32),
        ],
        scratch_types=[
            pltpu.VMEM((CPP * N1p,), jnp.float32),
            pltpu.VMEM((QCH * L,), jnp.int32),
            pltpu.VMEM((QCH * L,), jnp.float32),
            pltpu.VMEM((CPP * Mp,), jnp.float32),
        ],
    )
    def upscatter(prob_hbm, kif_hbm, l2t_hbm, upt_hbm, buf, ck, cp, l2):
        cid = lax.axis_index("c")
        sid = lax.axis_index("s")
        wid = sid * NC + cid
        for p in range(2):
            gch = wid * CPW + p * CPP

            def ze_step(j, _):
                buf[pl.ds(j * L, L)] = jnp.zeros((L,), jnp.float32)
                return 0

            lax.fori_loop(0, (CPP * N1p) // L, ze_step, 0)
            pltpu.sync_copy(l2t_hbm.at[pl.ds(gch * Mp, CPP * Mp)], l2)

            def ch_step(c, _):
                cb = c * (QCH * L)
                pltpu.sync_copy(kif_hbm.at[pl.ds(cb, QCH * L)], ck)
                pltpu.sync_copy(
                    prob_hbm.at[pl.ds(cid * (Mp * L) + cb, QCH * L)], cp)

                def q_step(i, _):
                    for part in range(4):
                        ii = i * 4 + part
                        o = ii * L
                        kr = ck[pl.ds(o, L)]
                        pr = cp[pl.ds(o, L)]
                        qcol = jnp.full((L,), c * QCH + ii, jnp.int32)
                        for ch in range(CPP):
                            val = plsc.load_gather(l2, [qcol + ch * Mp])
                            plsc.addupdate_scatter(
                                buf, [kr + ch * N1p], pr * val)
                    return 0

                lax.fori_loop(0, QCH // 4, q_step, 0)
                return 0

            lax.fori_loop(0, NCH, ch_step, 0)
            for ch in range(CPP):
                pltpu.sync_copy(buf.at[pl.ds(ch * N1p, N1p)],
                                upt_hbm.at[pl.ds((gch + ch) * N1p, N1p)])

    return upscatter


def kernel(p1, x1, o1, p2, x2, o2, W1, b1, g1, be1, W2, b2, g2, be2,
           csW1, csb1, csg, csbe, csW2, csb2):
    n1 = p1.shape[0]
    m = p2.shape[0]
    C = x1.shape[1]
    Ci = x2.shape[1]

    PER = _rup(-(-m // NW), 8)
    Mp = PER * NW
    DSTP = _rup(_rup(n1, 2048) // NW, 8)
    N1p = DSTP * NW

    p1x = jnp.asarray(p1[:, 0], jnp.float32)
    p1y = jnp.asarray(p1[:, 1], jnp.float32)
    p1z = jnp.asarray(p1[:, 2], jnp.float32)
    p2Tp = jnp.pad(p2.T, ((0, 0), (0, Mp - m)))
    p2b = p2Tp.reshape(3, NW, PER).transpose(1, 0, 2).reshape(NW * 3 * PER)
    x2p = jnp.pad(x2, ((0, Mp - m), (0, 0)))
    V3p = jnp.pad(csW1[:3], ((0, 5), (0, 0)))
    w2row = csW2.reshape(1, Ci)

    lin1 = _lin1_call(x1, W1, b1, g1, be1)
    lin2T, A = _dense2_call(x2p, W2, b2, g2, be2, csW1[3:], csb1)

    knn = _make_knn(n1, PER, Mp)
    kif1, prx1, pry1, prz1 = knn(p1x, p1y, p1z, p2b)
    prx = prx1.reshape(Mp, L)
    pry = pry1.reshape(Mp, L)
    prz = prz1.reshape(Mp, L)

    shrink = _shrink_call(A, prx, pry, prz, V3p, csg, csbe, w2row, csb2)

    shrink1 = shrink.reshape(Mp * L)
    partials, exs = _make_exden(n1, m, Mp, N1p)(shrink1, kif1)
    probs, = _make_probs(m, Mp, N1p)(partials, exs, kif1)
    upTf, = _make_upscatter(Mp, N1p, C)(probs, kif1, lin2T.reshape(C * Mp))
    upT = upTf.reshape(C, N1p)

    lin1p = jnp.pad(lin1, ((0, N1p - n1), (0, 0)))
    return _final_call(lin1p, upT)[:n1]
